# baseline (speedup 1.0000x reference)
import functools

import jax
import jax.numpy as jnp
from jax import lax
from jax.experimental import pallas as pl
from jax.experimental.pallas import tpu as pltpu
from jax.experimental.pallas import tpu_sc as plsc

_NC = 2
_NS = 16
_NW = _NC * _NS

_D1 = 16
_D2 = 8

_CJ = 125
_CI = 16


def _dense_mlp_call(xb, w1, bc, f1c, w2, b1, f2, b2, blk):
  B, N, F = xb.shape

  def body(x_ref, w1_ref, bc_ref, f1c_ref, w2_ref, b1_ref, f2_ref, b2_ref,
           out_ref):
    for b in range(B):
      X = x_ref[b]
      h = jnp.maximum(
          jnp.dot(X, w1_ref[...], preferred_element_type=jnp.float32)
          + bc_ref[...], 0.0)
      g = jnp.maximum(
          jnp.dot(h, f1c_ref[...], preferred_element_type=jnp.float32)
          + jnp.dot(X, w2_ref[...], preferred_element_type=jnp.float32)
          + b1_ref[...], 0.0)
      p = jnp.dot(g, f2_ref[...], preferred_element_type=jnp.float32) \
          + b2_ref[...]
      out_ref[:, 5 * b:5 * b + 5] = p
    out_ref[:, 10:11] = jnp.ones((blk, 1), jnp.float32)
    out_ref[:, 11:16] = jnp.zeros((blk, 5), jnp.float32)

  grid = (N // blk,)
  full = lambda a: pl.BlockSpec(a.shape, lambda i: (0,) * a.ndim)
  return pl.pallas_call(
      body,
      grid=grid,
      in_specs=[
          pl.BlockSpec((B, blk, F), lambda i: (0, i, 0)),
          full(w1), full(bc), full(f1c), full(w2), full(b1), full(f2), full(b2),
      ],
      out_specs=pl.BlockSpec((blk, _D1), lambda i: (i, 0)),
      out_shape=jax.ShapeDtypeStruct((N, _D1), jnp.float32),
  )(xb, w1, bc, f1c, w2, b1, f2, b2)


def _make_agg(N, E, D):
  C = _CI * _CJ
  H = C // 2
  nj = H // _CJ
  assert E % (_NW * C) == 0
  nch = E // (_NW * C)
  ew = E // _NW
  rt = (N // _NS + 7) // 8 * 8
  assert rt * (_NS - 1) + rt >= N and (N - rt) % 8 == 0
  mesh = plsc.VectorSubcoreMesh(core_axis_name="c", subcore_axis_name="s")

  @functools.partial(
      pl.kernel,
      out_type=jax.ShapeDtypeStruct((_NC, N, D), jnp.float32),
      mesh=mesh,
      scratch_types=[
          pltpu.VMEM((C,), jnp.int32),
          pltpu.VMEM((C,), jnp.int32),
          pltpu.VMEM((H, D), jnp.float32),
          pltpu.VMEM((H, D), jnp.float32),
          pltpu.VMEM_SHARED((N, D), jnp.float32),
          pltpu.SemaphoreType.DMA,
          pltpu.SemaphoreType.DMA,
          pltpu.SemaphoreType.DMA,
          pltpu.SemaphoreType.DMA,
      ],
      compiler_params=pltpu.CompilerParams(use_tc_tiling_on_sc=False),
  )
  def agg(ei_hbm, table_hbm, zeros_hbm, out_hbm,
          sidx, didx, rows0, rows1, accum, gsem0, gsem1, ssem0, ssem1):
    cid = lax.axis_index("c")
    sid = lax.axis_index("s")
    wid = sid * _NC + cid

    off = jnp.minimum(sid * rt, N - rt)
    pltpu.sync_copy(zeros_hbm, accum.at[pl.ds(off, rt)])
    plsc.subcore_barrier()

    base = wid * ew
    rows = (rows0, rows1)
    gsem = (gsem0, gsem1)
    ssem = (ssem0, ssem1)

    def sdrain(h):
      pltpu.make_async_copy(
          rows[h], accum.at[didx.at[pl.ds(h * H, H)]], ssem[h]).wait()

    def chunk(g, carry):
      @pl.when(g > 0)
      def _():
        sdrain(0)
        sdrain(1)
      r0 = base + g * C
      pltpu.sync_copy(ei_hbm.at[0, pl.ds(r0, C)], sidx)
      pltpu.sync_copy(ei_hbm.at[1, pl.ds(r0, C)], didx)
      for h in range(2):
        pltpu.async_copy(
            table_hbm.at[sidx.at[pl.ds(h * H, H)]], rows[h], gsem[h])
      for h in range(2):
        pltpu.make_async_copy(
            table_hbm.at[sidx.at[pl.ds(h * H, H)]], rows[h], gsem[h]).wait()
        pltpu.async_copy(
            rows[h], accum.at[didx.at[pl.ds(h * H, H)]], ssem[h], add=True)
      return carry

    lax.fori_loop(0, nch, chunk, 0)
    sdrain(0)
    sdrain(1)

    plsc.subcore_barrier()
    pltpu.sync_copy(accum.at[pl.ds(off, rt)],
                    out_hbm.at[cid, pl.ds(off, rt)])

  return agg


def _combine1_call(part, table, U, V, cb, blk):
  _, N, D = part.shape

  def body(p_ref, t_ref, u_ref, v_ref, cb_ref, out_ref):
    agg = p_ref[0] + p_ref[1]
    cnt = agg[:, 10:11]
    r = 1.0 / jnp.maximum(cnt, 1.0)
    mean = agg * r
    S = (jnp.dot(mean, u_ref[...], preferred_element_type=jnp.float32)
         + jnp.dot(t_ref[...], v_ref[...], preferred_element_type=jnp.float32)
         + cb_ref[...])
    out_ref[...] = S
    out_ref[:, 4:5] = r

  grid = (N // blk,)
  full = lambda a: pl.BlockSpec(a.shape, lambda i: (0,) * a.ndim)
  return pl.pallas_call(
      body,
      grid=grid,
      in_specs=[
          pl.BlockSpec((2, blk, D), lambda i: (0, i, 0)),
          pl.BlockSpec((blk, D), lambda i: (i, 0)),
          full(U), full(V), full(cb),
      ],
      out_specs=pl.BlockSpec((blk, _D2), lambda i: (i, 0)),
      out_shape=jax.ShapeDtypeStruct((N, _D2), jnp.float32),
  )(part, table, U, V, cb)


def _combine2_call(part2, stable, blk):
  _, N, D = part2.shape

  def body(p_ref, s_ref, out_ref):
    agg = p_ref[0] + p_ref[1]
    r = s_ref[:, 4:5]
    out_ref[...] = agg[:, 0:2] * r + s_ref[:, 2:4]

  grid = (N // blk,)
  return pl.pallas_call(
      body,
      grid=grid,
      in_specs=[
          pl.BlockSpec((2, blk, D), lambda i: (0, i, 0)),
          pl.BlockSpec((blk, D), lambda i: (i, 0)),
      ],
      out_specs=pl.BlockSpec((blk, 2), lambda i: (i, 0)),
      out_shape=jax.ShapeDtypeStruct((N, 2), jnp.float32),
  )(part2, stable)


def kernel(x, edge_index, ct_w, ct_b, cr_w, cr_b, ctp_w, ctp_b, cs_w, cs_b,
           fc1_w, fc1_b, fc2_w, fc2_b, g1_l_w, g1_l_b, g1_r_w,
           g2_l_w, g2_l_b, g2_r_w):
  B, S, N, F = x.shape
  E = edge_index.shape[1]
  f32 = jnp.float32

  OC = ct_w.shape[0]
  Wc = jnp.zeros((F - 12, 4 * OC), f32)
  for i, w in enumerate((ct_w, cr_w, ctp_w, cs_w)):
    Wc = Wc.at[24 * i:24 * (i + 1), OC * i:OC * (i + 1)].set(w[:, 0, :].T)
  W1 = jnp.zeros((F, 4 * OC), f32).at[12:].set(Wc)
  bc = jnp.concatenate([ct_b, cr_b, ctp_b, cs_b])[None, :]
  W2 = jnp.zeros((F, fc1_w.shape[0]), f32).at[:12].set(fc1_w[:, :12].T)
  F1c = fc1_w[:, 12:].T
  b1 = fc1_b[None, :]
  F2 = fc2_w.T
  b2 = fc2_b[None, :]

  L1 = g1_l_w.T
  R1 = g1_r_w.T
  L2 = g2_l_w.T
  R2 = g2_r_w.T
  u = (L1 @ L2)[:, 0]
  v = (R1 @ L2)[:, 0]
  u2 = (L1 @ R2)[:, 0]
  v2 = (R1 @ R2)[:, 0]
  c1 = (g1_l_b @ L2)[0]
  c2 = (g1_l_b @ R2)[0] + g2_l_b[0]
  U = jnp.zeros((_D1, _D2), f32)
  V = jnp.zeros((_D1, _D2), f32)
  for b in range(B):
    U = U.at[5 * b:5 * b + 5, b].set(u)
    U = U.at[5 * b:5 * b + 5, 2 + b].set(u2)
    V = V.at[5 * b:5 * b + 5, b].set(v)
    V = V.at[5 * b:5 * b + 5, 2 + b].set(v2)
  cb = jnp.zeros((1, _D2), f32)
  cb = cb.at[0, 0:2].set(c1).at[0, 2:4].set(c2)

  xb = x.reshape(B, N, F)
  table = _dense_mlp_call(xb, W1, bc, F1c, W2, b1, F2, b2, blk=5000)

  rt = (N // _NS + 7) // 8 * 8
  zeros1 = jnp.zeros((rt, _D1), f32)
  part1 = _make_agg(N, E, _D1)(edge_index, table, zeros1)

  stable = _combine1_call(part1, table, U, V, cb, blk=10000)

  zeros2 = jnp.zeros((rt, _D2), f32)
  part2 = _make_agg(N, E, _D2)(edge_index, stable, zeros2)

  out = _combine2_call(part2, stable, blk=10000)

  return out.T.reshape(B, 1, N)


# speedup vs baseline: 1.0017x; 1.0017x over previous
import functools

import jax
import jax.numpy as jnp
from jax import lax
from jax.experimental import pallas as pl
from jax.experimental.pallas import tpu as pltpu
from jax.experimental.pallas import tpu_sc as plsc

_NC = 2
_NS = 16
_NW = _NC * _NS

_D1 = 16
_D2 = 8

_CJ = 125
_CI = 16


def _dense_mlp_call(xb, w1, bc, f1c, w2, b1, f2, b2, blk):
  B, N, F = xb.shape

  def body(x_ref, w1_ref, bc_ref, f1c_ref, w2_ref, b1_ref, f2_ref, b2_ref,
           out_ref):
    for b in range(B):
      X = x_ref[b]
      h = jnp.maximum(
          jnp.dot(X, w1_ref[...], preferred_element_type=jnp.float32)
          + bc_ref[...], 0.0)
      g = jnp.maximum(
          jnp.dot(h, f1c_ref[...], preferred_element_type=jnp.float32)
          + jnp.dot(X, w2_ref[...], preferred_element_type=jnp.float32)
          + b1_ref[...], 0.0)
      p = jnp.dot(g, f2_ref[...], preferred_element_type=jnp.float32) \
          + b2_ref[...]
      out_ref[:, 5 * b:5 * b + 5] = p
    out_ref[:, 10:11] = jnp.ones((blk, 1), jnp.float32)
    out_ref[:, 11:16] = jnp.zeros((blk, 5), jnp.float32)

  grid = (N // blk,)
  full = lambda a: pl.BlockSpec(a.shape, lambda i: (0,) * a.ndim)
  return pl.pallas_call(
      body,
      grid=grid,
      in_specs=[
          pl.BlockSpec((B, blk, F), lambda i: (0, i, 0)),
          full(w1), full(bc), full(f1c), full(w2), full(b1), full(f2), full(b2),
      ],
      out_specs=pl.BlockSpec((blk, _D1), lambda i: (i, 0)),
      out_shape=jax.ShapeDtypeStruct((N, _D1), jnp.float32),
  )(xb, w1, bc, f1c, w2, b1, f2, b2)


def _edge_split_call(edge_index, blk=8192):
  _, E = edge_index.shape
  g = -(-E // blk)

  def body(e_ref, s_ref, d_ref):
    s_ref[...] = e_ref[0]
    d_ref[...] = e_ref[1]

  return pl.pallas_call(
      body,
      grid=(g,),
      in_specs=[pl.BlockSpec((2, blk), lambda i: (0, i))],
      out_specs=[pl.BlockSpec((blk,), lambda i: (i,)),
                 pl.BlockSpec((blk,), lambda i: (i,))],
      out_shape=[jax.ShapeDtypeStruct((g * blk,), jnp.int32)] * 2,
  )(edge_index)


def _make_agg(N, E, D):
  C = _CI * _CJ
  H = C // 2
  nj = H // _CJ
  assert E % (_NW * C) == 0
  nch = E // (_NW * C)
  ew = E // _NW
  rt = (N // _NS + 7) // 8 * 8
  assert rt * (_NS - 1) + rt >= N and (N - rt) % 8 == 0
  mesh = plsc.VectorSubcoreMesh(core_axis_name="c", subcore_axis_name="s")

  @functools.partial(
      pl.kernel,
      out_type=jax.ShapeDtypeStruct((_NC, N, D), jnp.float32),
      mesh=mesh,
      scratch_types=[
          pltpu.VMEM((C,), jnp.int32),
          pltpu.VMEM((C,), jnp.int32),
          pltpu.VMEM((H, D), jnp.float32),
          pltpu.VMEM((H, D), jnp.float32),
          pltpu.VMEM_SHARED((N, D), jnp.float32),
          pltpu.SemaphoreType.DMA,
          pltpu.SemaphoreType.DMA,
          pltpu.SemaphoreType.DMA,
          pltpu.SemaphoreType.DMA,
      ],
      compiler_params=pltpu.CompilerParams(use_tc_tiling_on_sc=False),
  )
  def agg(src_hbm, dst_hbm, table_hbm, zeros_hbm, out_hbm,
          sidx, didx, rows0, rows1, accum, gsem0, gsem1, ssem0, ssem1):
    cid = lax.axis_index("c")
    sid = lax.axis_index("s")
    wid = sid * _NC + cid

    off = jnp.minimum(sid * rt, N - rt)
    pltpu.sync_copy(zeros_hbm, accum.at[pl.ds(off, rt)])
    plsc.subcore_barrier()

    base = wid * ew
    rows = (rows0, rows1)
    gsem = (gsem0, gsem1)
    ssem = (ssem0, ssem1)

    def sdrain(h):
      pltpu.make_async_copy(
          rows[h], accum.at[didx.at[pl.ds(h * H, H)]], ssem[h]).wait()

    def chunk(g, carry):
      @pl.when(g > 0)
      def _():
        sdrain(0)
        sdrain(1)
      r0 = base + g * C
      pltpu.sync_copy(src_hbm.at[pl.ds(r0, C)], sidx)
      pltpu.sync_copy(dst_hbm.at[pl.ds(r0, C)], didx)
      for h in range(2):
        pltpu.async_copy(
            table_hbm.at[sidx.at[pl.ds(h * H, H)]], rows[h], gsem[h])
      for h in range(2):
        pltpu.make_async_copy(
            table_hbm.at[sidx.at[pl.ds(h * H, H)]], rows[h], gsem[h]).wait()
        pltpu.async_copy(
            rows[h], accum.at[didx.at[pl.ds(h * H, H)]], ssem[h], add=True)
      return carry

    lax.fori_loop(0, nch, chunk, 0)
    sdrain(0)
    sdrain(1)

    plsc.subcore_barrier()
    pltpu.sync_copy(accum.at[pl.ds(off, rt)],
                    out_hbm.at[cid, pl.ds(off, rt)])

  return agg


def _combine1_call(part, table, U, V, cb, blk):
  _, N, D = part.shape

  def body(p_ref, t_ref, u_ref, v_ref, cb_ref, out_ref):
    agg = p_ref[0] + p_ref[1]
    cnt = agg[:, 10:11]
    r = 1.0 / jnp.maximum(cnt, 1.0)
    mean = agg * r
    S = (jnp.dot(mean, u_ref[...], preferred_element_type=jnp.float32)
         + jnp.dot(t_ref[...], v_ref[...], preferred_element_type=jnp.float32)
         + cb_ref[...])
    out_ref[...] = S
    out_ref[:, 4:5] = r

  grid = (N // blk,)
  full = lambda a: pl.BlockSpec(a.shape, lambda i: (0,) * a.ndim)
  return pl.pallas_call(
      body,
      grid=grid,
      in_specs=[
          pl.BlockSpec((2, blk, D), lambda i: (0, i, 0)),
          pl.BlockSpec((blk, D), lambda i: (i, 0)),
          full(U), full(V), full(cb),
      ],
      out_specs=pl.BlockSpec((blk, _D2), lambda i: (i, 0)),
      out_shape=jax.ShapeDtypeStruct((N, _D2), jnp.float32),
  )(part, table, U, V, cb)


def _combine2_call(part2, stable, blk):
  _, N, D = part2.shape

  def body(p_ref, s_ref, out_ref):
    agg = p_ref[0] + p_ref[1]
    r = s_ref[:, 4:5]
    out_ref[...] = agg[:, 0:2] * r + s_ref[:, 2:4]

  grid = (N // blk,)
  return pl.pallas_call(
      body,
      grid=grid,
      in_specs=[
          pl.BlockSpec((2, blk, D), lambda i: (0, i, 0)),
          pl.BlockSpec((blk, D), lambda i: (i, 0)),
      ],
      out_specs=pl.BlockSpec((blk, 2), lambda i: (i, 0)),
      out_shape=jax.ShapeDtypeStruct((N, 2), jnp.float32),
  )(part2, stable)


def kernel(x, edge_index, ct_w, ct_b, cr_w, cr_b, ctp_w, ctp_b, cs_w, cs_b,
           fc1_w, fc1_b, fc2_w, fc2_b, g1_l_w, g1_l_b, g1_r_w,
           g2_l_w, g2_l_b, g2_r_w):
  B, S, N, F = x.shape
  E = edge_index.shape[1]
  f32 = jnp.float32

  OC = ct_w.shape[0]
  Wc = jnp.zeros((F - 12, 4 * OC), f32)
  for i, w in enumerate((ct_w, cr_w, ctp_w, cs_w)):
    Wc = Wc.at[24 * i:24 * (i + 1), OC * i:OC * (i + 1)].set(w[:, 0, :].T)
  W1 = jnp.zeros((F, 4 * OC), f32).at[12:].set(Wc)
  bc = jnp.concatenate([ct_b, cr_b, ctp_b, cs_b])[None, :]
  W2 = jnp.zeros((F, fc1_w.shape[0]), f32).at[:12].set(fc1_w[:, :12].T)
  F1c = fc1_w[:, 12:].T
  b1 = fc1_b[None, :]
  F2 = fc2_w.T
  b2 = fc2_b[None, :]

  L1 = g1_l_w.T
  R1 = g1_r_w.T
  L2 = g2_l_w.T
  R2 = g2_r_w.T
  u = (L1 @ L2)[:, 0]
  v = (R1 @ L2)[:, 0]
  u2 = (L1 @ R2)[:, 0]
  v2 = (R1 @ R2)[:, 0]
  c1 = (g1_l_b @ L2)[0]
  c2 = (g1_l_b @ R2)[0] + g2_l_b[0]
  U = jnp.zeros((_D1, _D2), f32)
  V = jnp.zeros((_D1, _D2), f32)
  for b in range(B):
    U = U.at[5 * b:5 * b + 5, b].set(u)
    U = U.at[5 * b:5 * b + 5, 2 + b].set(u2)
    V = V.at[5 * b:5 * b + 5, b].set(v)
    V = V.at[5 * b:5 * b + 5, 2 + b].set(v2)
  cb = jnp.zeros((1, _D2), f32)
  cb = cb.at[0, 0:2].set(c1).at[0, 2:4].set(c2)

  xb = x.reshape(B, N, F)
  table = _dense_mlp_call(xb, W1, bc, F1c, W2, b1, F2, b2, blk=5000)

  src_flat, dst_flat = _edge_split_call(edge_index)
  rt = (N // _NS + 7) // 8 * 8
  zeros1 = jnp.zeros((rt, _D1), f32)
  part1 = _make_agg(N, E, _D1)(src_flat, dst_flat, table, zeros1)

  stable = _combine1_call(part1, table, U, V, cb, blk=10000)

  zeros2 = jnp.zeros((rt, _D2), f32)
  part2 = _make_agg(N, E, _D2)(src_flat, dst_flat, stable, zeros2)

  out = _combine2_call(part2, stable, blk=10000)

  return out.T.reshape(B, 1, N)


# speedup vs baseline: 1.2008x; 1.1988x over previous
import functools

import jax
import jax.numpy as jnp
from jax import lax
from jax.experimental import pallas as pl
from jax.experimental.pallas import tpu as pltpu
from jax.experimental.pallas import tpu_sc as plsc

_NC = 2
_NS = 16
_NW = _NC * _NS

_D1 = 16
_D2 = 16

_CJ = 125
_CI = 16


def _dense_mlp_call(xt, w1t, bc, f1ct, w2t, b1, f2t, b2, blk):
  B, F, N = xt.shape

  def body(x_ref, w1_ref, bc_ref, f1c_ref, w2_ref, b1_ref, f2_ref, b2_ref,
           out_ref):
    ps = []
    for b in range(B):
      X = x_ref[b]
      h = jnp.maximum(
          jnp.dot(w1_ref[...], X, preferred_element_type=jnp.float32)
          + bc_ref[...], 0.0)
      g = jnp.maximum(
          jnp.dot(f1c_ref[...], h, preferred_element_type=jnp.float32)
          + jnp.dot(w2_ref[...], X, preferred_element_type=jnp.float32)
          + b1_ref[...], 0.0)
      ps.append(jnp.dot(f2_ref[...], g, preferred_element_type=jnp.float32)
                + b2_ref[...])
    t16 = jnp.concatenate(
        [ps[0], ps[1],
         jnp.ones((1, blk), jnp.float32),
         jnp.zeros((5, blk), jnp.float32)], axis=0)
    out_ref[...] = t16.T

  grid = (pl.cdiv(N, blk),)
  full = lambda a: pl.BlockSpec(a.shape, lambda i: (0,) * a.ndim)
  return pl.pallas_call(
      body,
      grid=grid,
      in_specs=[
          pl.BlockSpec((B, F, blk), lambda i: (0, 0, i)),
          full(w1t), full(bc), full(f1ct), full(w2t), full(b1), full(f2t),
          full(b2),
      ],
      out_specs=pl.BlockSpec((blk, _D1), lambda i: (i, 0)),
      out_shape=jax.ShapeDtypeStruct((N, _D1), jnp.float32),
  )(xt, w1t, bc, f1ct, w2t, b1, f2t, b2)


def _edge_split_call(edge_index, blk=8192):
  _, E = edge_index.shape
  g = -(-E // blk)

  def body(e_ref, s_ref, d_ref):
    s_ref[...] = e_ref[0]
    d_ref[...] = e_ref[1]

  return pl.pallas_call(
      body,
      grid=(g,),
      in_specs=[pl.BlockSpec((2, blk), lambda i: (0, i))],
      out_specs=[pl.BlockSpec((blk,), lambda i: (i,)),
                 pl.BlockSpec((blk,), lambda i: (i,))],
      out_shape=[jax.ShapeDtypeStruct((g * blk,), jnp.int32)] * 2,
  )(edge_index)


def _make_agg(N, E, D):
  C = _CI * _CJ
  H = C // 2
  nj = H // _CJ
  assert E % (_NW * C) == 0
  nch = E // (_NW * C)
  ew = E // _NW
  rt = (N // _NS + 7) // 8 * 8
  assert rt * (_NS - 1) + rt >= N and (N - rt) % 8 == 0
  mesh = plsc.VectorSubcoreMesh(core_axis_name="c", subcore_axis_name="s")

  @functools.partial(
      pl.kernel,
      out_type=jax.ShapeDtypeStruct((_NC, N, D), jnp.float32),
      mesh=mesh,
      scratch_types=[
          pltpu.VMEM((C,), jnp.int32),
          pltpu.VMEM((C,), jnp.int32),
          pltpu.VMEM((H, D), jnp.float32),
          pltpu.VMEM((H, D), jnp.float32),
          pltpu.VMEM_SHARED((N, D), jnp.float32),
          pltpu.SemaphoreType.DMA,
          pltpu.SemaphoreType.DMA,
          pltpu.SemaphoreType.DMA,
          pltpu.SemaphoreType.DMA,
      ],
      compiler_params=pltpu.CompilerParams(use_tc_tiling_on_sc=False),
  )
  def agg(src_hbm, dst_hbm, table_hbm, zeros_hbm, out_hbm,
          sidx, didx, rows0, rows1, accum, gsem0, gsem1, ssem0, ssem1):
    cid = lax.axis_index("c")
    sid = lax.axis_index("s")
    wid = sid * _NC + cid

    off = jnp.minimum(sid * rt, N - rt)
    pltpu.sync_copy(zeros_hbm, accum.at[pl.ds(off, rt)])
    plsc.subcore_barrier()

    base = wid * ew
    rows = (rows0, rows1)
    gsem = (gsem0, gsem1)
    ssem = (ssem0, ssem1)

    def sdrain(h):
      pltpu.make_async_copy(
          rows[h], accum.at[didx.at[pl.ds(h * H, H)]], ssem[h]).wait()

    def chunk(g, carry):
      @pl.when(g > 0)
      def _():
        sdrain(0)
        sdrain(1)
      r0 = base + g * C
      pltpu.sync_copy(src_hbm.at[pl.ds(r0, C)], sidx)
      pltpu.sync_copy(dst_hbm.at[pl.ds(r0, C)], didx)
      for h in range(2):
        pltpu.async_copy(
            table_hbm.at[sidx.at[pl.ds(h * H, H)]], rows[h], gsem[h])
      for h in range(2):
        pltpu.make_async_copy(
            table_hbm.at[sidx.at[pl.ds(h * H, H)]], rows[h], gsem[h]).wait()
        pltpu.async_copy(
            rows[h], accum.at[didx.at[pl.ds(h * H, H)]], ssem[h], add=True)
      return carry

    lax.fori_loop(0, nch, chunk, 0)
    sdrain(0)
    sdrain(1)

    plsc.subcore_barrier()
    pltpu.sync_copy(accum.at[pl.ds(off, rt)],
                    out_hbm.at[cid, pl.ds(off, rt)])

  return agg


def _combine1_call(part, table, UU, VV, CB, cbias, rmask, blk):
  _, R, _ = part.shape

  def body(p_ref, t_ref, uu_ref, vv_ref, cbm_ref, cb_ref, rm_ref, out_ref):
    agg = p_ref[0] + p_ref[1]
    cntb = jnp.dot(agg, cbm_ref[...], preferred_element_type=jnp.float32)
    r = 1.0 / jnp.maximum(cntb, 1.0)
    S = (jnp.dot(agg, uu_ref[...], preferred_element_type=jnp.float32) * r
         + jnp.dot(t_ref[...], vv_ref[...], preferred_element_type=jnp.float32)
         + cb_ref[...] + r * rm_ref[...])
    out_ref[...] = S

  grid = (R // blk,)
  full = lambda a: pl.BlockSpec(a.shape, lambda i: (0,) * a.ndim)
  return pl.pallas_call(
      body,
      grid=grid,
      in_specs=[
          pl.BlockSpec((2, blk, 128), lambda i: (0, i, 0)),
          pl.BlockSpec((blk, 128), lambda i: (i, 0)),
          full(UU), full(VV), full(CB), full(cbias), full(rmask),
      ],
      out_specs=pl.BlockSpec((blk, 128), lambda i: (i, 0)),
      out_shape=jax.ShapeDtypeStruct((R, 128), jnp.float32),
  )(part, table, UU, VV, CB, cbias, rmask)


def _combine2_call(part2, stable, CB4, QM, m01, blk):
  _, R, _ = part2.shape

  def body(p_ref, s_ref, cb4_ref, qm_ref, m01_ref, out_ref):
    agg = p_ref[0] + p_ref[1]
    rb = jnp.dot(s_ref[...], cb4_ref[...], preferred_element_type=jnp.float32)
    qm = jnp.dot(s_ref[...], qm_ref[...], preferred_element_type=jnp.float32)
    out_ref[...] = agg * rb * m01_ref[...] + qm

  grid = (R // blk,)
  full = lambda a: pl.BlockSpec(a.shape, lambda i: (0,) * a.ndim)
  return pl.pallas_call(
      body,
      grid=grid,
      in_specs=[
          pl.BlockSpec((2, blk, 128), lambda i: (0, i, 0)),
          pl.BlockSpec((blk, 128), lambda i: (i, 0)),
          full(CB4), full(QM), full(m01),
      ],
      out_specs=pl.BlockSpec((blk, 128), lambda i: (i, 0)),
      out_shape=jax.ShapeDtypeStruct((R, 128), jnp.float32),
  )(part2, stable, CB4, QM, m01)


def kernel(x, edge_index, ct_w, ct_b, cr_w, cr_b, ctp_w, ctp_b, cs_w, cs_b,
           fc1_w, fc1_b, fc2_w, fc2_b, g1_l_w, g1_l_b, g1_r_w,
           g2_l_w, g2_l_b, g2_r_w):
  B, S, N, F = x.shape
  E = edge_index.shape[1]
  f32 = jnp.float32

  OC = ct_w.shape[0]
  NH = 4 * OC
  Wc = jnp.zeros((F - 12, NH), f32)
  for i, w in enumerate((ct_w, cr_w, ctp_w, cs_w)):
    Wc = Wc.at[24 * i:24 * (i + 1), OC * i:OC * (i + 1)].set(w[:, 0, :].T)
  W1t = jnp.zeros((NH, F), f32).at[:, 12:].set(Wc.T)
  bc = jnp.concatenate([ct_b, cr_b, ctp_b, cs_b])[:, None]
  W2t = jnp.zeros((fc1_w.shape[0], F), f32).at[:, :12].set(fc1_w[:, :12])
  F1ct = fc1_w[:, 12:]
  b1 = fc1_b[:, None]
  F2t = fc2_w
  b2 = fc2_b[:, None]

  L1 = g1_l_w.T
  R1 = g1_r_w.T
  L2 = g2_l_w.T
  R2 = g2_r_w.T
  u = (L1 @ L2)[:, 0]
  v = (R1 @ L2)[:, 0]
  u2 = (L1 @ R2)[:, 0]
  v2 = (R1 @ R2)[:, 0]
  c1 = (g1_l_b @ L2)[0]
  c2 = (g1_l_b @ R2)[0] + g2_l_b[0]
  U16 = jnp.zeros((16, 16), f32)
  V16 = jnp.zeros((16, 16), f32)
  for b in range(B):
    U16 = U16.at[5 * b:5 * b + 5, b].set(u)
    U16 = U16.at[5 * b:5 * b + 5, 2 + b].set(u2)
    V16 = V16.at[5 * b:5 * b + 5, b].set(v)
    V16 = V16.at[5 * b:5 * b + 5, 2 + b].set(v2)
  eye8 = jnp.eye(8, dtype=f32)
  UU = jnp.kron(eye8, U16)
  VV = jnp.kron(eye8, V16)
  CB = jnp.kron(eye8, jnp.zeros((16, 16), f32).at[10, :].set(1.0))
  CB4 = jnp.kron(eye8, jnp.zeros((16, 16), f32).at[4, :].set(1.0))
  QM = jnp.kron(eye8, jnp.zeros((16, 16), f32).at[2, 0].set(1.0)
                .at[3, 1].set(1.0))
  cb16 = jnp.zeros((16,), f32).at[0:2].set(c1).at[2:4].set(c2)
  cbias = jnp.tile(cb16, 8)[None, :]
  rmask = jnp.tile(jnp.zeros((16,), f32).at[4].set(1.0), 8)[None, :]
  m01 = jnp.tile(jnp.zeros((16,), f32).at[0:2].set(1.0), 8)[None, :]

  xt = jnp.transpose(x, (0, 3, 1, 2)).reshape(B, F, N)
  table = _dense_mlp_call(xt, W1t, bc, F1ct, W2t, b1, F2t, b2, blk=6272)

  src_flat, dst_flat = _edge_split_call(edge_index)
  rt = (N // _NS + 7) // 8 * 8
  zeros1 = jnp.zeros((rt, _D1), f32)
  part1 = _make_agg(N, E, _D1)(src_flat, dst_flat, table, zeros1)

  R = N // 8
  stable128 = _combine1_call(part1.reshape(2, R, 128), table.reshape(R, 128),
                             UU, VV, CB, cbias, rmask, blk=6250)
  stable = stable128.reshape(N, _D2)

  part2 = _make_agg(N, E, _D2)(src_flat, dst_flat, stable, zeros1)

  out128 = _combine2_call(part2.reshape(2, R, 128), stable128,
                          CB4, QM, m01, blk=6250)

  hv = out128.reshape(N, 16)[:, :2]
  return hv.T.reshape(B, 1, N)


# speedup vs baseline: 1.4978x; 1.2473x over previous
import functools

import jax
import jax.numpy as jnp
from jax import lax
from jax.experimental import pallas as pl
from jax.experimental.pallas import tpu as pltpu
from jax.experimental.pallas import tpu_sc as plsc

_NC = 2
_NS = 16
_NW = _NC * _NS

_D1 = 16
_D2 = 16

_CJ = 125
_CI = 16


def _dense_mlp_call(xt, w1t, bc, f1ct, w2t, b1, f2t, b2, blk):
  BF, _, N = xt.shape
  F = BF // 2
  B = 2

  def body(x_ref, w1_ref, bc_ref, f1c_ref, w2_ref, b1_ref, f2_ref, b2_ref,
           out_ref):
    ps = []
    for b in range(B):
      X = x_ref[b * F:(b + 1) * F, 0, :]
      h = jnp.maximum(
          jnp.dot(w1_ref[...], X, preferred_element_type=jnp.float32)
          + bc_ref[...], 0.0)
      g = jnp.maximum(
          jnp.dot(f1c_ref[...], h, preferred_element_type=jnp.float32)
          + jnp.dot(w2_ref[...], X, preferred_element_type=jnp.float32)
          + b1_ref[...], 0.0)
      ps.append(jnp.dot(f2_ref[...], g, preferred_element_type=jnp.float32)
                + b2_ref[...])
    t16 = jnp.concatenate(
        [ps[0], ps[1],
         jnp.ones((1, blk), jnp.float32),
         jnp.zeros((5, blk), jnp.float32)], axis=0)
    out_ref[...] = t16.T

  grid = (pl.cdiv(N, blk),)
  full = lambda a: pl.BlockSpec(a.shape, lambda i: (0,) * a.ndim)
  return pl.pallas_call(
      body,
      grid=grid,
      in_specs=[
          pl.BlockSpec((BF, 1, blk), lambda i: (0, 0, i)),
          full(w1t), full(bc), full(f1ct), full(w2t), full(b1), full(f2t),
          full(b2),
      ],
      out_specs=pl.BlockSpec((blk, _D1), lambda i: (i, 0)),
      out_shape=jax.ShapeDtypeStruct((N, _D1), jnp.float32),
  )(xt, w1t, bc, f1ct, w2t, b1, f2t, b2)


def _edge_split_call(edge_index, blk=65536):
  _, E = edge_index.shape
  g = -(-E // blk)

  def body(e_ref, s_ref, d_ref):
    s_ref[...] = e_ref[0]
    d_ref[...] = e_ref[1]

  return pl.pallas_call(
      body,
      grid=(g,),
      in_specs=[pl.BlockSpec((2, blk), lambda i: (0, i))],
      out_specs=[pl.BlockSpec((blk,), lambda i: (i,)),
                 pl.BlockSpec((blk,), lambda i: (i,))],
      out_shape=[jax.ShapeDtypeStruct((g * blk,), jnp.int32)] * 2,
  )(edge_index)


def _make_agg(N, E, D):
  C = _CI * _CJ
  H = C // 2
  nj = H // _CJ
  assert E % (_NW * C) == 0
  nch = E // (_NW * C)
  ew = E // _NW
  rt = (N // _NS + 7) // 8 * 8
  assert rt * (_NS - 1) + rt >= N and (N - rt) % 8 == 0
  mesh = plsc.VectorSubcoreMesh(core_axis_name="c", subcore_axis_name="s")

  @functools.partial(
      pl.kernel,
      out_type=jax.ShapeDtypeStruct((_NC, N, D), jnp.float32),
      mesh=mesh,
      scratch_types=[
          pltpu.VMEM((C,), jnp.int32),
          pltpu.VMEM((C,), jnp.int32),
          pltpu.VMEM((H, D), jnp.float32),
          pltpu.VMEM((H, D), jnp.float32),
          pltpu.VMEM_SHARED((N, D), jnp.float32),
          pltpu.SemaphoreType.DMA,
          pltpu.SemaphoreType.DMA,
          pltpu.SemaphoreType.DMA,
          pltpu.SemaphoreType.DMA,
      ],
      compiler_params=pltpu.CompilerParams(use_tc_tiling_on_sc=False),
  )
  def agg(src_hbm, dst_hbm, table_hbm, zeros_hbm, out_hbm,
          sidx, didx, rows0, rows1, accum, gsem0, gsem1, ssem0, ssem1):
    cid = lax.axis_index("c")
    sid = lax.axis_index("s")
    wid = sid * _NC + cid

    off = jnp.minimum(sid * rt, N - rt)
    pltpu.sync_copy(zeros_hbm, accum.at[pl.ds(off, rt)])
    plsc.subcore_barrier()

    base = wid * ew
    rows = (rows0, rows1)
    gsem = (gsem0, gsem1)
    ssem = (ssem0, ssem1)

    def sdrain(h):
      pltpu.make_async_copy(
          rows[h], accum.at[didx.at[pl.ds(h * H, H)]], ssem[h]).wait()

    def chunk(g, carry):
      @pl.when(g > 0)
      def _():
        sdrain(0)
        sdrain(1)
      r0 = base + g * C
      pltpu.sync_copy(src_hbm.at[pl.ds(r0, C)], sidx)
      pltpu.sync_copy(dst_hbm.at[pl.ds(r0, C)], didx)
      for h in range(2):
        pltpu.async_copy(
            table_hbm.at[sidx.at[pl.ds(h * H, H)]], rows[h], gsem[h])
      for h in range(2):
        pltpu.make_async_copy(
            table_hbm.at[sidx.at[pl.ds(h * H, H)]], rows[h], gsem[h]).wait()
        pltpu.async_copy(
            rows[h], accum.at[didx.at[pl.ds(h * H, H)]], ssem[h], add=True)
      return carry

    lax.fori_loop(0, nch, chunk, 0)
    sdrain(0)
    sdrain(1)

    plsc.subcore_barrier()
    pltpu.sync_copy(accum.at[pl.ds(off, rt)],
                    out_hbm.at[cid, pl.ds(off, rt)])

  return agg


def _combine1_call(part, table, UU, VV, CB, cbias, rmask, blk):
  _, R, _ = part.shape

  def body(p_ref, t_ref, uu_ref, vv_ref, cbm_ref, cb_ref, rm_ref, out_ref):
    agg = p_ref[0] + p_ref[1]
    cntb = jnp.dot(agg, cbm_ref[...], preferred_element_type=jnp.float32)
    r = 1.0 / jnp.maximum(cntb, 1.0)
    S = (jnp.dot(agg, uu_ref[...], preferred_element_type=jnp.float32) * r
         + jnp.dot(t_ref[...], vv_ref[...], preferred_element_type=jnp.float32)
         + cb_ref[...] + r * rm_ref[...])
    out_ref[...] = S

  grid = (R // blk,)
  full = lambda a: pl.BlockSpec(a.shape, lambda i: (0,) * a.ndim)
  return pl.pallas_call(
      body,
      grid=grid,
      in_specs=[
          pl.BlockSpec((2, blk, 128), lambda i: (0, i, 0)),
          pl.BlockSpec((blk, 128), lambda i: (i, 0)),
          full(UU), full(VV), full(CB), full(cbias), full(rmask),
      ],
      out_specs=pl.BlockSpec((blk, 128), lambda i: (i, 0)),
      out_shape=jax.ShapeDtypeStruct((R, 128), jnp.float32),
  )(part, table, UU, VV, CB, cbias, rmask)


def _combine2_call(part2, stable, CB4, QM, m01, blk):
  _, R, _ = part2.shape

  def body(p_ref, s_ref, cb4_ref, qm_ref, m01_ref, out_ref):
    agg = p_ref[0] + p_ref[1]
    rb = jnp.dot(s_ref[...], cb4_ref[...], preferred_element_type=jnp.float32)
    qm = jnp.dot(s_ref[...], qm_ref[...], preferred_element_type=jnp.float32)
    out_ref[...] = agg * rb * m01_ref[...] + qm

  grid = (R // blk,)
  full = lambda a: pl.BlockSpec(a.shape, lambda i: (0,) * a.ndim)
  return pl.pallas_call(
      body,
      grid=grid,
      in_specs=[
          pl.BlockSpec((2, blk, 128), lambda i: (0, i, 0)),
          pl.BlockSpec((blk, 128), lambda i: (i, 0)),
          full(CB4), full(QM), full(m01),
      ],
      out_specs=pl.BlockSpec((blk, 128), lambda i: (i, 0)),
      out_shape=jax.ShapeDtypeStruct((R, 128), jnp.float32),
  )(part2, stable, CB4, QM, m01)


def kernel(x, edge_index, ct_w, ct_b, cr_w, cr_b, ctp_w, ctp_b, cs_w, cs_b,
           fc1_w, fc1_b, fc2_w, fc2_b, g1_l_w, g1_l_b, g1_r_w,
           g2_l_w, g2_l_b, g2_r_w):
  B, S, N, F = x.shape
  E = edge_index.shape[1]
  f32 = jnp.float32

  OC = ct_w.shape[0]
  NH = 4 * OC
  Wc = jnp.zeros((F - 12, NH), f32)
  for i, w in enumerate((ct_w, cr_w, ctp_w, cs_w)):
    Wc = Wc.at[24 * i:24 * (i + 1), OC * i:OC * (i + 1)].set(w[:, 0, :].T)
  W1t = jnp.zeros((NH, F), f32).at[:, 12:].set(Wc.T)
  bc = jnp.concatenate([ct_b, cr_b, ctp_b, cs_b])[:, None]
  W2t = jnp.zeros((fc1_w.shape[0], F), f32).at[:, :12].set(fc1_w[:, :12])
  F1ct = fc1_w[:, 12:]
  b1 = fc1_b[:, None]
  F2t = fc2_w
  b2 = fc2_b[:, None]

  L1 = g1_l_w.T
  R1 = g1_r_w.T
  L2 = g2_l_w.T
  R2 = g2_r_w.T
  u = (L1 @ L2)[:, 0]
  v = (R1 @ L2)[:, 0]
  u2 = (L1 @ R2)[:, 0]
  v2 = (R1 @ R2)[:, 0]
  c1 = (g1_l_b @ L2)[0]
  c2 = (g1_l_b @ R2)[0] + g2_l_b[0]
  U16 = jnp.zeros((16, 16), f32)
  V16 = jnp.zeros((16, 16), f32)
  for b in range(B):
    U16 = U16.at[5 * b:5 * b + 5, b].set(u)
    U16 = U16.at[5 * b:5 * b + 5, 2 + b].set(u2)
    V16 = V16.at[5 * b:5 * b + 5, b].set(v)
    V16 = V16.at[5 * b:5 * b + 5, 2 + b].set(v2)
  eye8 = jnp.eye(8, dtype=f32)
  UU = jnp.kron(eye8, U16)
  VV = jnp.kron(eye8, V16)
  CB = jnp.kron(eye8, jnp.zeros((16, 16), f32).at[10, :].set(1.0))
  CB4 = jnp.kron(eye8, jnp.zeros((16, 16), f32).at[4, :].set(1.0))
  QM = jnp.kron(eye8, jnp.zeros((16, 16), f32).at[2, 0].set(1.0)
                .at[3, 1].set(1.0))
  cb16 = jnp.zeros((16,), f32).at[0:2].set(c1).at[2:4].set(c2)
  cbias = jnp.tile(cb16, 8)[None, :]
  rmask = jnp.tile(jnp.zeros((16,), f32).at[4].set(1.0), 8)[None, :]
  m01 = jnp.tile(jnp.zeros((16,), f32).at[0:2].set(1.0), 8)[None, :]

  xt = jnp.transpose(x, (0, 3, 1, 2)).reshape(B * F, 1, N)
  table = _dense_mlp_call(xt, W1t, bc, F1ct, W2t, b1, F2t, b2, blk=6272)

  src_flat, dst_flat = _edge_split_call(edge_index)
  rt = (N // _NS + 7) // 8 * 8
  zeros1 = jnp.zeros((rt, _D1), f32)
  part1 = _make_agg(N, E, _D1)(src_flat, dst_flat, table, zeros1)

  R = N // 8
  stable128 = _combine1_call(part1.reshape(2, R, 128), table.reshape(R, 128),
                             UU, VV, CB, cbias, rmask, blk=6250)
  stable = stable128.reshape(N, _D2)

  part2 = _make_agg(N, E, _D2)(src_flat, dst_flat, stable, zeros1)

  out128 = _combine2_call(part2.reshape(2, R, 128), stable128,
                          CB4, QM, m01, blk=6250)

  hv = out128.reshape(N, 16)[:, :2]
  return hv.T.reshape(B, 1, N)
